# baseline (speedup 1.0000x reference)
import functools

import jax
import jax.numpy as jnp
from jax import lax
from jax.experimental import pallas as pl
from jax.experimental.pallas import tpu as pltpu
from jax.experimental.pallas import tpu_sc as plsc

SEQ = 32768
DIM_HALF = 64
PACKED = 2 * DIM_HALF
TAB_ROWS = 9216

_info = plsc.get_sparse_core_info()
_NC, _NS = _info.num_cores, _info.num_subcores
_NW = _NC * _NS

_SPLITS = 4
_SEQ_SPLIT = SEQ // _SPLITS
_BPW = _SEQ_SPLIT // _NW


def _make_gather(split):
  mesh = plsc.VectorSubcoreMesh(core_axis_name="c", subcore_axis_name="s")

  @functools.partial(
      pl.kernel,
      mesh=mesh,
      compiler_params=pltpu.CompilerParams(use_tc_tiling_on_sc=True),
      out_type=jax.ShapeDtypeStruct((_SEQ_SPLIT, PACKED), jnp.float32),
      scratch_types=[
          pltpu.VMEM((_BPW,), jnp.int32),
          pltpu.VMEM((_BPW, PACKED), jnp.float32),
          pltpu.SemaphoreType.DMA,
      ],
      name=f"rope_gather_{split}",
  )
  def rope_gather(pos_hbm, tab_hbm, out_hbm, idx_v, buf, gsem):
    wid = lax.axis_index("s") * _NC + lax.axis_index("c")
    base = wid * _BPW
    pltpu.sync_copy(pos_hbm.at[pl.ds(split * _SEQ_SPLIT + base, _BPW)],
                    idx_v)
    pltpu.async_copy(tab_hbm.at[idx_v], buf, gsem).wait()
    pltpu.sync_copy(buf, out_hbm.at[pl.ds(base, _BPW)])

  return rope_gather


_gathers = [_make_gather(s) for s in range(_SPLITS)]

_PACK_BLK = 2304


def _pack_body(cos_t_ref, sin_t_ref, out_ref):
  out_ref[...] = jnp.concatenate(
      [cos_t_ref[...].T, sin_t_ref[...].T], axis=1)


_pack_t = pl.pallas_call(
    _pack_body,
    grid=(TAB_ROWS // _PACK_BLK,),
    in_specs=[
        pl.BlockSpec((DIM_HALF, _PACK_BLK), lambda i: (0, i)),
        pl.BlockSpec((DIM_HALF, _PACK_BLK), lambda i: (0, i)),
    ],
    out_specs=pl.BlockSpec((_PACK_BLK, PACKED), lambda i: (i, 0)),
    out_shape=jax.ShapeDtypeStruct((TAB_ROWS, PACKED), jnp.float32),
)

_UNPACK_BLK = 2048
_BLKS_PER_SPLIT = _SEQ_SPLIT // _UNPACK_BLK


def _unpack_first_body(packed_ref, cos_ref, sin_ref):
  xt = packed_ref[...].T
  cos_ref[...] = xt[:DIM_HALF, :]
  sin_ref[...] = xt[DIM_HALF:, :]


def _unpack_rest_body(cos_acc_ref, sin_acc_ref, packed_ref,
                      cos_ref, sin_ref):
  del cos_acc_ref, sin_acc_ref
  xt = packed_ref[...].T
  cos_ref[...] = xt[:DIM_HALF, :]
  sin_ref[...] = xt[DIM_HALF:, :]


def _make_unpack(split):
  out_specs = [
      pl.BlockSpec((DIM_HALF, _UNPACK_BLK),
                   lambda i: (0, split * _BLKS_PER_SPLIT + i)),
      pl.BlockSpec((DIM_HALF, _UNPACK_BLK),
                   lambda i: (0, split * _BLKS_PER_SPLIT + i)),
  ]
  out_shape = [
      jax.ShapeDtypeStruct((DIM_HALF, SEQ), jnp.float32),
      jax.ShapeDtypeStruct((DIM_HALF, SEQ), jnp.float32),
  ]
  packed_spec = pl.BlockSpec((_UNPACK_BLK, PACKED), lambda i: (i, 0))
  if split == 0:
    return pl.pallas_call(
        _unpack_first_body,
        grid=(_BLKS_PER_SPLIT,),
        in_specs=[packed_spec],
        out_specs=out_specs,
        out_shape=out_shape,
    )
  return pl.pallas_call(
      _unpack_rest_body,
      grid=(_BLKS_PER_SPLIT,),
      in_specs=[
          pl.BlockSpec(memory_space=pl.ANY),
          pl.BlockSpec(memory_space=pl.ANY),
          packed_spec,
      ],
      out_specs=out_specs,
      out_shape=out_shape,
      input_output_aliases={0: 0, 1: 1},
  )


_unpacks = [_make_unpack(s) for s in range(_SPLITS)]


@jax.jit
def kernel(positions, cos_cache, sin_cache):
  pos = positions.astype(jnp.int32)
  packed_tab = _pack_t(cos_cache.T, sin_cache.T)
  packed = [_gathers[s](pos, packed_tab) for s in range(_SPLITS)]
  cos_t, sin_t = _unpacks[0](packed[0])
  for s in range(1, _SPLITS):
    cos_t, sin_t = _unpacks[s](cos_t, sin_t, packed[s])
  return (cos_t.T, sin_t.T)


# speedup vs baseline: 1.1006x; 1.1006x over previous
import functools

import jax
import jax.numpy as jnp
from jax import lax
from jax.experimental import pallas as pl
from jax.experimental.pallas import tpu as pltpu
from jax.experimental.pallas import tpu_sc as plsc

SEQ = 32768
DIM_HALF = 64
PACKED = 2 * DIM_HALF
TAB_ROWS = 9216

_info = plsc.get_sparse_core_info()
_NC, _NS = _info.num_cores, _info.num_subcores
_NW = _NC * _NS

_SPLITS = 2
_SEQ_SPLIT = SEQ // _SPLITS
_BPW = _SEQ_SPLIT // _NW
_CHUNK = 256
_NCH = _BPW // _CHUNK


def _make_gather(split):
  mesh = plsc.VectorSubcoreMesh(core_axis_name="c", subcore_axis_name="s")

  @functools.partial(
      pl.kernel,
      mesh=mesh,
      compiler_params=pltpu.CompilerParams(use_tc_tiling_on_sc=True),
      out_type=jax.ShapeDtypeStruct((_SEQ_SPLIT, PACKED), jnp.float32),
      scratch_types=[
          pltpu.VMEM((_BPW,), jnp.int32),
          pltpu.VMEM((_CHUNK, PACKED), jnp.float32),
          pltpu.VMEM((_CHUNK, PACKED), jnp.float32),
          pltpu.SemaphoreType.DMA,
          pltpu.SemaphoreType.DMA,
          pltpu.SemaphoreType.DMA,
          pltpu.SemaphoreType.DMA,
      ],
      name=f"rope_gather_{split}",
  )
  def rope_gather(pos_hbm, tab_hbm, out_hbm,
                  idx_v, buf0, buf1, gsem0, gsem1, wsem0, wsem1):
    wid = lax.axis_index("s") * _NC + lax.axis_index("c")
    base = wid * _BPW
    pltpu.sync_copy(pos_hbm.at[pl.ds(split * _SEQ_SPLIT + base, _BPW)],
                    idx_v)
    buf = (buf0, buf1)
    gsem = (gsem0, gsem1)
    wsem = (wsem0, wsem1)

    def gather(c):
      p = c % 2
      idx_c = idx_v.at[pl.ds(c * _CHUNK, _CHUNK)]
      return pltpu.async_copy(tab_hbm.at[idx_c], buf[p], gsem[p])

    def write(c):
      p = c % 2
      return pltpu.async_copy(
          buf[p], out_hbm.at[pl.ds(base + c * _CHUNK, _CHUNK)], wsem[p])

    pending_g = [None, None]
    pending_w = [None, None]
    pending_g[0] = gather(0)
    for c in range(_NCH):
      p = c % 2
      p1 = (c + 1) % 2
      if c + 1 < _NCH:
        if pending_w[p1] is not None:
          pending_w[p1].wait()
          pending_w[p1] = None
        pending_g[p1] = gather(c + 1)
      pending_g[p].wait()
      pending_w[p] = write(c)
    for p in range(2):
      if pending_w[p] is not None:
        pending_w[p].wait()

  return rope_gather


_gathers = [_make_gather(s) for s in range(_SPLITS)]

_PACK_BLK = 2304


def _pack_body(cos_t_ref, sin_t_ref, out_ref):
  out_ref[...] = jnp.concatenate(
      [cos_t_ref[...].T, sin_t_ref[...].T], axis=1)


_pack_t = pl.pallas_call(
    _pack_body,
    grid=(TAB_ROWS // _PACK_BLK,),
    in_specs=[
        pl.BlockSpec((DIM_HALF, _PACK_BLK), lambda i: (0, i)),
        pl.BlockSpec((DIM_HALF, _PACK_BLK), lambda i: (0, i)),
    ],
    out_specs=pl.BlockSpec((_PACK_BLK, PACKED), lambda i: (i, 0)),
    out_shape=jax.ShapeDtypeStruct((TAB_ROWS, PACKED), jnp.float32),
)

_UNPACK_BLK = 2048
_BLKS_PER_SPLIT = _SEQ_SPLIT // _UNPACK_BLK


def _unpack_first_body(packed_ref, cos_ref, sin_ref):
  xt = packed_ref[...].T
  cos_ref[...] = xt[:DIM_HALF, :]
  sin_ref[...] = xt[DIM_HALF:, :]


def _unpack_rest_body(cos_acc_ref, sin_acc_ref, packed_ref,
                      cos_ref, sin_ref):
  del cos_acc_ref, sin_acc_ref
  xt = packed_ref[...].T
  cos_ref[...] = xt[:DIM_HALF, :]
  sin_ref[...] = xt[DIM_HALF:, :]


def _make_unpack(split):
  out_specs = [
      pl.BlockSpec((DIM_HALF, _UNPACK_BLK),
                   lambda i: (0, split * _BLKS_PER_SPLIT + i)),
      pl.BlockSpec((DIM_HALF, _UNPACK_BLK),
                   lambda i: (0, split * _BLKS_PER_SPLIT + i)),
  ]
  out_shape = [
      jax.ShapeDtypeStruct((DIM_HALF, SEQ), jnp.float32),
      jax.ShapeDtypeStruct((DIM_HALF, SEQ), jnp.float32),
  ]
  packed_spec = pl.BlockSpec((_UNPACK_BLK, PACKED), lambda i: (i, 0))
  if split == 0:
    return pl.pallas_call(
        _unpack_first_body,
        grid=(_BLKS_PER_SPLIT,),
        in_specs=[packed_spec],
        out_specs=out_specs,
        out_shape=out_shape,
    )
  return pl.pallas_call(
      _unpack_rest_body,
      grid=(_BLKS_PER_SPLIT,),
      in_specs=[
          pl.BlockSpec(memory_space=pl.ANY),
          pl.BlockSpec(memory_space=pl.ANY),
          packed_spec,
      ],
      out_specs=out_specs,
      out_shape=out_shape,
      input_output_aliases={0: 0, 1: 1},
  )


_unpacks = [_make_unpack(s) for s in range(_SPLITS)]


@jax.jit
def kernel(positions, cos_cache, sin_cache):
  pos = positions.astype(jnp.int32)
  packed_tab = _pack_t(cos_cache.T, sin_cache.T)
  packed = [_gathers[s](pos, packed_tab) for s in range(_SPLITS)]
  cos_t, sin_t = _unpacks[0](packed[0])
  for s in range(1, _SPLITS):
    cos_t, sin_t = _unpacks[s](cos_t, sin_t, packed[s])
  return (cos_t.T, sin_t.T)
